# Initial kernel scaffold; baseline (speedup 1.0000x reference)
import jax
import jax.numpy as jnp
from jax.experimental import pallas as pl


def kernel(node_h, coords, edge_index, edge_feat, W_in, b_in, W_e1_0, b_e1_0, W_e2_0, b_e2_0, W_n1_0, b_n1_0, W_n2_0, b_n2_0, W_e1_1, b_e1_1, W_e2_1, b_e2_1, W_n1_1, b_n1_1, W_n2_1, b_n2_1, W_out, b_out, Wc):
    raise NotImplementedError("write your pallas kernel here")


# speedup vs baseline: 2.8029x; 2.8029x over previous
import functools

import jax
import jax.numpy as jnp
from jax import lax
from jax.experimental import pallas as pl
from jax.experimental.pallas import tpu as pltpu
from jax.experimental.pallas import tpu_sc as plsc

N = 100000
E = 1600000
C = 32
D_IN = 102
G = 128
NG = E // G
NC, NS = 2, 16
NW = NC * NS
GP_W = NG // NW
GP_W_REM = NG - GP_W * NW
GP_S = NG // NS
GP_S_REM = NG - GP_S * NS
RPS = N // NS
ZR = 125

_f32 = jnp.float32
_bf16 = jnp.bfloat16

_SC_PARAMS = pltpu.CompilerParams(
    needs_layout_passes=False, use_tc_tiling_on_sc=False)


@functools.cache
def _mesh():
    return plsc.VectorSubcoreMesh(
        core_axis_name="c", subcore_axis_name="s",
        num_cores=NC, num_subcores=NS)


def _silu(x):
    return x * jax.nn.sigmoid(x)


def _bdot(a, b):
    return jnp.dot(a.astype(_bf16), b.astype(_bf16),
                   preferred_element_type=_f32)


def _sqd_body(cp_hbm, src2_hbm, dst2_hbm, sqd_hbm,
              idxd_v, idxs_v, rd_v, rs_v, out_v, semd, sems):
    wid = lax.axis_index("s") * NC + lax.axis_index("c")
    base = wid * GP_W + jnp.minimum(wid, GP_W_REM)
    ng = GP_W + jnp.where(wid < GP_W_REM, 1, 0)
    lanes = lax.iota(jnp.int32, 16)

    @pl.loop(0, ng)
    def _(i):
        g = base + i
        pltpu.sync_copy(dst2_hbm.at[g], idxd_v)
        pltpu.sync_copy(src2_hbm.at[g], idxs_v)
        cd = pltpu.async_copy(cp_hbm.at[idxd_v], rd_v, semd)
        cs = pltpu.async_copy(cp_hbm.at[idxs_v], rs_v, sems)
        cd.wait()
        cs.wait()
        for t in range(8):
            rows = t * 16 + lanes
            acc = jnp.zeros((16,), _f32)
            for l in range(3):
                col = jnp.full((16,), l, jnp.int32)
                vd = plsc.load_gather(rd_v, [rows, col])
                vs = plsc.load_gather(rs_v, [rows, col])
                d = vd - vs
                acc = acc + d * d
            out_v[pl.ds(t * 16, 16)] = acc
        pltpu.sync_copy(out_v, sqd_hbm.at[pl.ds(g * G, G)])


@jax.jit
def _sc_sqd(cp, src2, dst2):
    return pl.kernel(
        _sqd_body,
        out_type=jax.ShapeDtypeStruct((E,), _f32),
        mesh=_mesh(),
        scratch_types=[
            pltpu.VMEM((G,), jnp.int32),
            pltpu.VMEM((G,), jnp.int32),
            pltpu.VMEM((G, 16), _f32),
            pltpu.VMEM((G, 16), _f32),
            pltpu.VMEM((G,), _f32),
            pltpu.SemaphoreType.DMA,
            pltpu.SemaphoreType.DMA,
        ],
        compiler_params=_SC_PARAMS,
    )(cp, src2, dst2)


def _gath_body(hb_hbm, src2_hbm, dst2_hbm, hd_hbm, hs_hbm,
               idxd_v, idxs_v, bufd_v, bufs_v, semd, sems):
    wid = lax.axis_index("s") * NC + lax.axis_index("c")
    base = wid * GP_W + jnp.minimum(wid, GP_W_REM)
    ng = GP_W + jnp.where(wid < GP_W_REM, 1, 0)

    @pl.loop(0, ng)
    def _(i):
        g = base + i
        pltpu.sync_copy(dst2_hbm.at[g], idxd_v)
        pltpu.sync_copy(src2_hbm.at[g], idxs_v)
        cd = pltpu.async_copy(hb_hbm.at[idxd_v], bufd_v, semd)
        cs = pltpu.async_copy(hb_hbm.at[idxs_v], bufs_v, sems)
        cd.wait()
        cs.wait()
        pltpu.sync_copy(bufd_v, hd_hbm.at[pl.ds(g * G, G), :])
        pltpu.sync_copy(bufs_v, hs_hbm.at[pl.ds(g * G, G), :])


@jax.jit
def _sc_gather(hb, src2, dst2):
    return pl.kernel(
        _gath_body,
        out_type=(jax.ShapeDtypeStruct((E, C), _bf16),
                  jax.ShapeDtypeStruct((E, C), _bf16)),
        mesh=_mesh(),
        scratch_types=[
            pltpu.VMEM((G,), jnp.int32),
            pltpu.VMEM((G,), jnp.int32),
            pltpu.VMEM((G, C), _bf16),
            pltpu.VMEM((G, C), _bf16),
            pltpu.SemaphoreType.DMA,
            pltpu.SemaphoreType.DMA,
        ],
        compiler_params=_SC_PARAMS,
    )(hb, src2, dst2)


def _scat_body(m_hbm, dst2_hbm, agg_hbm, idx_v, mb_v, zb_v, acc_sh):
    c = lax.axis_index("c")
    s = lax.axis_index("s")
    cc = c * 16

    @pl.loop(0, ZR)
    def _(r):
        zb_v[r, :] = jnp.zeros((16,), _f32)

    @pl.loop(0, RPS // ZR)
    def _(j):
        pltpu.sync_copy(zb_v, acc_sh.at[pl.ds(s * RPS + j * ZR, ZR), :])

    plsc.subcore_barrier()

    base = s * GP_S + jnp.minimum(s, GP_S_REM)
    ng = GP_S + jnp.where(s < GP_S_REM, 1, 0)

    @pl.loop(0, ng)
    def _(i):
        g = base + i
        pltpu.sync_copy(dst2_hbm.at[g], idx_v)
        pltpu.sync_copy(m_hbm.at[pl.ds(g * G, G), pl.ds(cc, 16)], mb_v)
        pltpu.sync_copy(mb_v, acc_sh.at[idx_v], add=True)

    plsc.subcore_barrier()

    @pl.loop(0, RPS // ZR)
    def _(j):
        r0 = s * RPS + j * ZR
        pltpu.sync_copy(acc_sh.at[pl.ds(r0, ZR), :],
                        agg_hbm.at[pl.ds(r0, ZR), pl.ds(cc, 16)])


@jax.jit
def _sc_scatter(m, dst2):
    return pl.kernel(
        _scat_body,
        out_type=jax.ShapeDtypeStruct((N, C), _f32),
        mesh=_mesh(),
        scratch_types=[
            pltpu.VMEM((G,), jnp.int32),
            pltpu.VMEM((G, 16), _f32),
            pltpu.VMEM((ZR, 16), _f32),
            pltpu.VMEM_SHARED((N, 16), _f32),
        ],
        compiler_params=_SC_PARAMS,
    )(m, dst2)


BN = 1000
BE = 4000


def _full_spec(shape):
    return pl.BlockSpec(shape, lambda i: tuple(0 for _ in shape))


def _row_spec(rows, cols):
    return pl.BlockSpec((rows, cols), lambda i: (i, 0))


def _pro_body(nh, co, Win, bin_, hb_o, cp_o):
    h = _silu(_bdot(nh[...], Win[...]) + bin_[...])
    hb_o[...] = h.astype(_bf16)
    cp_o[...] = jnp.concatenate(
        [co[...], jnp.zeros((BN, 13), _f32)], axis=1)


@jax.jit
def _tc_pro(node_h, coords, W_in, b_in):
    return pl.pallas_call(
        _pro_body,
        grid=(N // BN,),
        in_specs=[
            _row_spec(BN, D_IN), _row_spec(BN, 3),
            _full_spec((D_IN, C)), _full_spec((1, C)),
        ],
        out_specs=[_row_spec(BN, C), _row_spec(BN, 16)],
        out_shape=[jax.ShapeDtypeStruct((N, C), _bf16),
                   jax.ShapeDtypeStruct((N, 16), _f32)],
        compiler_params=pltpu.CompilerParams(
            dimension_semantics=("arbitrary",)),
    )(node_h, coords, W_in, b_in)


def _mid_body(hd, hs, sq, ef, We1hh, wc, Wf, b1, We2, b2, m_o):
    x = jnp.concatenate([hd[...], hs[...]], axis=1)
    sqb = sq[...].astype(_bf16).astype(_f32)
    wcb = wc[...].astype(_bf16).astype(_f32)
    a = (jnp.dot(x, We1hh[...].astype(_bf16), preferred_element_type=_f32)
         + sqb * wcb
         + _bdot(ef[...], Wf[...])
         + b1[...])
    m1 = _silu(a)
    z = _bdot(m1, We2[...]) + b2[...]
    m_o[...] = _silu(z)


@jax.jit
def _tc_mid(hd, hs, sq2, ef, We1hh, wc, Wf, b1, We2, b2):
    return pl.pallas_call(
        _mid_body,
        grid=(E // BE,),
        in_specs=[
            _row_spec(BE, C), _row_spec(BE, C), _row_spec(BE, 1),
            _row_spec(BE, 3),
            _full_spec((2 * C, C)), _full_spec((1, C)), _full_spec((3, C)),
            _full_spec((1, C)),
            _full_spec((C, C)), _full_spec((1, C)),
        ],
        out_specs=[_row_spec(BE, C)],
        out_shape=[jax.ShapeDtypeStruct((E, C), _f32)],
        compiler_params=pltpu.CompilerParams(
            dimension_semantics=("arbitrary",)),
    )(hd, hs, sq2, ef, We1hh, wc, Wf, b1, We2, b2)[0]


def _node_body(hb, agg, Wn1, bn1, Wn2, bn2, hb_o):
    hn = jnp.concatenate([hb[...], agg[...].astype(_bf16)], axis=1)
    t = jnp.dot(hn, Wn1[...].astype(_bf16),
                preferred_element_type=_f32) + bn1[...]
    h2 = _bdot(_silu(t), Wn2[...]) + bn2[...]
    hb_o[...] = h2.astype(_bf16)


@jax.jit
def _tc_node(hb, agg, Wn1, bn1, Wn2, bn2):
    return pl.pallas_call(
        _node_body,
        grid=(N // BN,),
        in_specs=[
            _row_spec(BN, C), _row_spec(BN, C),
            _full_spec((2 * C, C)), _full_spec((1, C)),
            _full_spec((C, C)), _full_spec((1, C)),
        ],
        out_specs=[_row_spec(BN, C)],
        out_shape=[jax.ShapeDtypeStruct((N, C), _bf16)],
        compiler_params=pltpu.CompilerParams(
            dimension_semantics=("arbitrary",)),
    )(hb, agg, Wn1, bn1, Wn2, bn2)[0]


def _fin_body(hb, agg, Wn1, bn1, Wn2, bn2, Wout, bout, Wc, hs_o, cs_o):
    hn = jnp.concatenate([hb[...], agg[...].astype(_bf16)], axis=1)
    t = jnp.dot(hn, Wn1[...].astype(_bf16),
                preferred_element_type=_f32) + bn1[...]
    h2 = _bdot(_silu(t), Wn2[...]) + bn2[...]
    hs = _silu(_bdot(h2, Wout[...]) + bout[...])
    hs_o[...] = hs
    cs_o[...] = _bdot(hs, Wc[...])


@jax.jit
def _tc_fin(hb, agg, Wn1, bn1, Wn2, bn2, Wout, bout, Wc):
    return pl.pallas_call(
        _fin_body,
        grid=(N // BN,),
        in_specs=[
            _row_spec(BN, C), _row_spec(BN, C),
            _full_spec((2 * C, C)), _full_spec((1, C)),
            _full_spec((C, C)), _full_spec((1, C)),
            _full_spec((C, C)), _full_spec((1, C)),
            _full_spec((C, 6)),
        ],
        out_specs=[_row_spec(BN, C), _row_spec(BN, 6)],
        out_shape=[jax.ShapeDtypeStruct((N, C), _f32),
                   jax.ShapeDtypeStruct((N, 6), _f32)],
        compiler_params=pltpu.CompilerParams(
            dimension_semantics=("arbitrary",)),
    )(hb, agg, Wn1, bn1, Wn2, bn2, Wout, bout, Wc)


def kernel(node_h, coords, edge_index, edge_feat,
           W_in, b_in,
           W_e1_0, b_e1_0, W_e2_0, b_e2_0, W_n1_0, b_n1_0, W_n2_0, b_n2_0,
           W_e1_1, b_e1_1, W_e2_1, b_e2_1, W_n1_1, b_n1_1, W_n2_1, b_n2_1,
           W_out, b_out, Wc):
    src2 = edge_index[0].reshape(NG, G)
    dst2 = edge_index[1].reshape(NG, G)

    r1 = lambda b: b.reshape(1, C)
    We1hh = (W_e1_0[:2 * C], W_e1_1[:2 * C])
    wc = (W_e1_0[2 * C:2 * C + 1], W_e1_1[2 * C:2 * C + 1])
    Wf = (W_e1_0[2 * C + 1:], W_e1_1[2 * C + 1:])
    b1 = (r1(b_e1_0), r1(b_e1_1))
    We2 = (W_e2_0, W_e2_1)
    b2 = (r1(b_e2_0), r1(b_e2_1))
    Wn1 = (W_n1_0, W_n1_1)
    bn1 = (r1(b_n1_0), r1(b_n1_1))
    Wn2 = (W_n2_0, W_n2_1)
    bn2 = (r1(b_n2_0), r1(b_n2_1))

    hb, cp = _tc_pro(node_h, coords, W_in, r1(b_in))
    sqd = _sc_sqd(cp, src2, dst2)
    sq2 = sqd.reshape(E, 1)

    hd, hs = _sc_gather(hb, src2, dst2)
    m = _tc_mid(hd, hs, sq2, edge_feat, We1hh[0], wc[0], Wf[0], b1[0],
                We2[0], b2[0])
    agg = _sc_scatter(m, dst2)
    hb = _tc_node(hb, agg, Wn1[0], bn1[0], Wn2[0], bn2[0])

    hd, hs = _sc_gather(hb, src2, dst2)
    m = _tc_mid(hd, hs, sq2, edge_feat, We1hh[1], wc[1], Wf[1], b1[1],
                We2[1], b2[1])
    agg = _sc_scatter(m, dst2)
    hs0, cs = _tc_fin(hb, agg, Wn1[1], bn1[1], Wn2[1], bn2[1],
                      W_out, r1(b_out), Wc)
    return hs0, cs


# speedup vs baseline: 3.5674x; 1.2727x over previous
import functools

import jax
import jax.numpy as jnp
from jax import lax
from jax.experimental import pallas as pl
from jax.experimental.pallas import tpu as pltpu
from jax.experimental.pallas import tpu_sc as plsc

N = 100000
E = 1600000
C = 32
D_IN = 102
G = 128
NG = E // G
NC, NS = 2, 16
NW = NC * NS
GP_W = NG // NW
GP_W_REM = NG - GP_W * NW
GP_S = NG // NS
GP_S_REM = NG - GP_S * NS
RPS = N // NS
ZR = 125
KB = 8
NBW = GP_W // KB
NBS = GP_S // KB

_f32 = jnp.float32
_bf16 = jnp.bfloat16

_SC_PARAMS = pltpu.CompilerParams(
    needs_layout_passes=False, use_tc_tiling_on_sc=False)


@functools.cache
def _mesh():
    return plsc.VectorSubcoreMesh(
        core_axis_name="c", subcore_axis_name="s",
        num_cores=NC, num_subcores=NS)


def _silu(x):
    return x * jax.nn.sigmoid(x)


def _bdot(a, b):
    return jnp.dot(a.astype(_bf16), b.astype(_bf16),
                   preferred_element_type=_f32)


def _sqd_body(cp_hbm, src2_hbm, dst2_hbm, sqd_hbm,
              idxd_v, idxs_v, rd_v, rs_v, out_v, semd, sems):
    wid = lax.axis_index("s") * NC + lax.axis_index("c")
    base = wid * GP_W + jnp.minimum(wid, GP_W_REM)
    ng = GP_W + jnp.where(wid < GP_W_REM, 1, 0)
    lanes = lax.iota(jnp.int32, 16)

    def _compute(b):
        for t in range(8):
            rows = b * G + t * 16 + lanes
            acc = jnp.zeros((16,), _f32)
            for l in range(3):
                col = jnp.full((16,), l, jnp.int32)
                vd = plsc.load_gather(rd_v, [rows, col])
                vs = plsc.load_gather(rs_v, [rows, col])
                d = vd - vs
                acc = acc + d * d
            out_v[pl.ds(b * G + t * 16, 16)] = acc

    @pl.loop(0, NBW)
    def _(i):
        gb = base + i * KB
        pltpu.sync_copy(dst2_hbm.at[pl.ds(gb, KB), :], idxd_v)
        pltpu.sync_copy(src2_hbm.at[pl.ds(gb, KB), :], idxs_v)
        ds_ = []
        for b in range(KB):
            ds_.append(pltpu.async_copy(
                cp_hbm.at[idxd_v.at[b]],
                rd_v.at[pl.ds(b * G, G), :], semd))
            ds_.append(pltpu.async_copy(
                cp_hbm.at[idxs_v.at[b]],
                rs_v.at[pl.ds(b * G, G), :], sems))
        for d in ds_:
            d.wait()
        for b in range(KB):
            _compute(b)
        pltpu.sync_copy(out_v, sqd_hbm.at[pl.ds(gb * G, KB * G)])

    @pl.loop(NBW * KB, ng)
    def _(j):
        g = base + j
        pltpu.sync_copy(dst2_hbm.at[g], idxd_v.at[0])
        pltpu.sync_copy(src2_hbm.at[g], idxs_v.at[0])
        cd = pltpu.async_copy(cp_hbm.at[idxd_v.at[0]],
                              rd_v.at[pl.ds(0, G), :], semd)
        cs = pltpu.async_copy(cp_hbm.at[idxs_v.at[0]],
                              rs_v.at[pl.ds(0, G), :], sems)
        cd.wait()
        cs.wait()
        _compute(0)
        pltpu.sync_copy(out_v.at[pl.ds(0, G)], sqd_hbm.at[pl.ds(g * G, G)])


@jax.jit
def _sc_sqd(cp, src2, dst2):
    return pl.kernel(
        _sqd_body,
        out_type=jax.ShapeDtypeStruct((E,), _f32),
        mesh=_mesh(),
        scratch_types=[
            pltpu.VMEM((KB, G), jnp.int32),
            pltpu.VMEM((KB, G), jnp.int32),
            pltpu.VMEM((KB * G, 16), _f32),
            pltpu.VMEM((KB * G, 16), _f32),
            pltpu.VMEM((KB * G,), _f32),
            pltpu.SemaphoreType.DMA,
            pltpu.SemaphoreType.DMA,
        ],
        compiler_params=_SC_PARAMS,
    )(cp, src2, dst2)


def _gath_body(hb_hbm, src2_hbm, dst2_hbm, hd_hbm, hs_hbm,
               idxd_v, idxs_v, bufd_v, bufs_v, semd, sems):
    wid = lax.axis_index("s") * NC + lax.axis_index("c")
    base = wid * GP_W + jnp.minimum(wid, GP_W_REM)
    ng = GP_W + jnp.where(wid < GP_W_REM, 1, 0)

    @pl.loop(0, NBW)
    def _(i):
        gb = base + i * KB
        pltpu.sync_copy(dst2_hbm.at[pl.ds(gb, KB), :], idxd_v)
        pltpu.sync_copy(src2_hbm.at[pl.ds(gb, KB), :], idxs_v)
        ds_ = []
        for b in range(KB):
            ds_.append(pltpu.async_copy(
                hb_hbm.at[idxd_v.at[b]],
                bufd_v.at[pl.ds(b * G, G), :], semd))
            ds_.append(pltpu.async_copy(
                hb_hbm.at[idxs_v.at[b]],
                bufs_v.at[pl.ds(b * G, G), :], sems))
        for d in ds_:
            d.wait()
        s1 = pltpu.async_copy(bufd_v, hd_hbm.at[pl.ds(gb * G, KB * G), :],
                              semd)
        s2 = pltpu.async_copy(bufs_v, hs_hbm.at[pl.ds(gb * G, KB * G), :],
                              sems)
        s1.wait()
        s2.wait()

    @pl.loop(NBW * KB, ng)
    def _(j):
        g = base + j
        pltpu.sync_copy(dst2_hbm.at[g], idxd_v.at[0])
        pltpu.sync_copy(src2_hbm.at[g], idxs_v.at[0])
        cd = pltpu.async_copy(hb_hbm.at[idxd_v.at[0]],
                              bufd_v.at[pl.ds(0, G), :], semd)
        cs = pltpu.async_copy(hb_hbm.at[idxs_v.at[0]],
                              bufs_v.at[pl.ds(0, G), :], sems)
        cd.wait()
        cs.wait()
        pltpu.sync_copy(bufd_v.at[pl.ds(0, G), :],
                        hd_hbm.at[pl.ds(g * G, G), :])
        pltpu.sync_copy(bufs_v.at[pl.ds(0, G), :],
                        hs_hbm.at[pl.ds(g * G, G), :])


@jax.jit
def _sc_gather(hb, src2, dst2):
    return pl.kernel(
        _gath_body,
        out_type=(jax.ShapeDtypeStruct((E, C), _bf16),
                  jax.ShapeDtypeStruct((E, C), _bf16)),
        mesh=_mesh(),
        scratch_types=[
            pltpu.VMEM((KB, G), jnp.int32),
            pltpu.VMEM((KB, G), jnp.int32),
            pltpu.VMEM((KB * G, C), _bf16),
            pltpu.VMEM((KB * G, C), _bf16),
            pltpu.SemaphoreType.DMA,
            pltpu.SemaphoreType.DMA,
        ],
        compiler_params=_SC_PARAMS,
    )(hb, src2, dst2)


def _scat_body(m_hbm, dst2_hbm, agg_hbm, idx_v, mb_v, zb_v, acc_sh, sem):
    c = lax.axis_index("c")
    s = lax.axis_index("s")
    cc = c * 16

    @pl.loop(0, ZR)
    def _(r):
        zb_v[r, :] = jnp.zeros((16,), _f32)

    @pl.loop(0, RPS // ZR)
    def _(j):
        pltpu.sync_copy(zb_v, acc_sh.at[pl.ds(s * RPS + j * ZR, ZR), :])

    plsc.subcore_barrier()

    base = s * GP_S + jnp.minimum(s, GP_S_REM)
    ng = GP_S + jnp.where(s < GP_S_REM, 1, 0)

    @pl.loop(0, NBS)
    def _(i):
        gb = base + i * KB
        pltpu.sync_copy(dst2_hbm.at[pl.ds(gb, KB), :], idx_v)
        pltpu.sync_copy(m_hbm.at[pl.ds(gb * G, KB * G), pl.ds(cc, 16)],
                        mb_v)
        ds_ = []
        for b in range(KB):
            ds_.append(pltpu.async_copy(
                mb_v.at[pl.ds(b * G, G), :],
                acc_sh.at[idx_v.at[b]], sem, add=True))
        for d in ds_:
            d.wait()

    @pl.loop(NBS * KB, ng)
    def _(j):
        g = base + j
        pltpu.sync_copy(dst2_hbm.at[g], idx_v.at[0])
        pltpu.sync_copy(m_hbm.at[pl.ds(g * G, G), pl.ds(cc, 16)],
                        mb_v.at[pl.ds(0, G), :])
        pltpu.sync_copy(mb_v.at[pl.ds(0, G), :], acc_sh.at[idx_v.at[0]],
                        add=True)

    plsc.subcore_barrier()

    @pl.loop(0, RPS // ZR)
    def _(j):
        r0 = s * RPS + j * ZR
        pltpu.sync_copy(acc_sh.at[pl.ds(r0, ZR), :],
                        agg_hbm.at[pl.ds(r0, ZR), pl.ds(cc, 16)])


@jax.jit
def _sc_scatter(m, dst2):
    return pl.kernel(
        _scat_body,
        out_type=jax.ShapeDtypeStruct((N, C), _f32),
        mesh=_mesh(),
        scratch_types=[
            pltpu.VMEM((KB, G), jnp.int32),
            pltpu.VMEM((KB * G, 16), _f32),
            pltpu.VMEM((ZR, 16), _f32),
            pltpu.VMEM_SHARED((N, 16), _f32),
            pltpu.SemaphoreType.DMA,
        ],
        compiler_params=_SC_PARAMS,
    )(m, dst2)


BN = 1000
BE = 4000


def _full_spec(shape):
    return pl.BlockSpec(shape, lambda i: tuple(0 for _ in shape))


def _row_spec(rows, cols):
    return pl.BlockSpec((rows, cols), lambda i: (i, 0))


def _pro_body(nh, co, Win, bin_, hb_o, cp_o):
    h = _silu(_bdot(nh[...], Win[...]) + bin_[...])
    hb_o[...] = h.astype(_bf16)
    cp_o[...] = jnp.concatenate(
        [co[...], jnp.zeros((BN, 13), _f32)], axis=1)


@jax.jit
def _tc_pro(node_h, coords, W_in, b_in):
    return pl.pallas_call(
        _pro_body,
        grid=(N // BN,),
        in_specs=[
            _row_spec(BN, D_IN), _row_spec(BN, 3),
            _full_spec((D_IN, C)), _full_spec((1, C)),
        ],
        out_specs=[_row_spec(BN, C), _row_spec(BN, 16)],
        out_shape=[jax.ShapeDtypeStruct((N, C), _bf16),
                   jax.ShapeDtypeStruct((N, 16), _f32)],
        compiler_params=pltpu.CompilerParams(
            dimension_semantics=("arbitrary",)),
    )(node_h, coords, W_in, b_in)


def _mid_body(hd, hs, sq, ef, We1hh, wc, Wf, b1, We2, b2, m_o):
    x = jnp.concatenate([hd[...], hs[...]], axis=1)
    sqb = sq[...].astype(_bf16).astype(_f32)
    wcb = wc[...].astype(_bf16).astype(_f32)
    a = (jnp.dot(x, We1hh[...].astype(_bf16), preferred_element_type=_f32)
         + sqb * wcb
         + _bdot(ef[...], Wf[...])
         + b1[...])
    m1 = _silu(a)
    z = _bdot(m1, We2[...]) + b2[...]
    m_o[...] = _silu(z)


@jax.jit
def _tc_mid(hd, hs, sq2, ef, We1hh, wc, Wf, b1, We2, b2):
    return pl.pallas_call(
        _mid_body,
        grid=(E // BE,),
        in_specs=[
            _row_spec(BE, C), _row_spec(BE, C), _row_spec(BE, 1),
            _row_spec(BE, 3),
            _full_spec((2 * C, C)), _full_spec((1, C)), _full_spec((3, C)),
            _full_spec((1, C)),
            _full_spec((C, C)), _full_spec((1, C)),
        ],
        out_specs=[_row_spec(BE, C)],
        out_shape=[jax.ShapeDtypeStruct((E, C), _f32)],
        compiler_params=pltpu.CompilerParams(
            dimension_semantics=("arbitrary",)),
    )(hd, hs, sq2, ef, We1hh, wc, Wf, b1, We2, b2)[0]


def _node_body(hb, agg, Wn1, bn1, Wn2, bn2, hb_o):
    hn = jnp.concatenate([hb[...], agg[...].astype(_bf16)], axis=1)
    t = jnp.dot(hn, Wn1[...].astype(_bf16),
                preferred_element_type=_f32) + bn1[...]
    h2 = _bdot(_silu(t), Wn2[...]) + bn2[...]
    hb_o[...] = h2.astype(_bf16)


@jax.jit
def _tc_node(hb, agg, Wn1, bn1, Wn2, bn2):
    return pl.pallas_call(
        _node_body,
        grid=(N // BN,),
        in_specs=[
            _row_spec(BN, C), _row_spec(BN, C),
            _full_spec((2 * C, C)), _full_spec((1, C)),
            _full_spec((C, C)), _full_spec((1, C)),
        ],
        out_specs=[_row_spec(BN, C)],
        out_shape=[jax.ShapeDtypeStruct((N, C), _bf16)],
        compiler_params=pltpu.CompilerParams(
            dimension_semantics=("arbitrary",)),
    )(hb, agg, Wn1, bn1, Wn2, bn2)[0]


def _fin_body(hb, agg, Wn1, bn1, Wn2, bn2, Wout, bout, Wc, hs_o, cs_o):
    hn = jnp.concatenate([hb[...], agg[...].astype(_bf16)], axis=1)
    t = jnp.dot(hn, Wn1[...].astype(_bf16),
                preferred_element_type=_f32) + bn1[...]
    h2 = _bdot(_silu(t), Wn2[...]) + bn2[...]
    hs = _silu(_bdot(h2, Wout[...]) + bout[...])
    hs_o[...] = hs
    cs_o[...] = _bdot(hs, Wc[...])


@jax.jit
def _tc_fin(hb, agg, Wn1, bn1, Wn2, bn2, Wout, bout, Wc):
    return pl.pallas_call(
        _fin_body,
        grid=(N // BN,),
        in_specs=[
            _row_spec(BN, C), _row_spec(BN, C),
            _full_spec((2 * C, C)), _full_spec((1, C)),
            _full_spec((C, C)), _full_spec((1, C)),
            _full_spec((C, C)), _full_spec((1, C)),
            _full_spec((C, 6)),
        ],
        out_specs=[_row_spec(BN, C), _row_spec(BN, 6)],
        out_shape=[jax.ShapeDtypeStruct((N, C), _f32),
                   jax.ShapeDtypeStruct((N, 6), _f32)],
        compiler_params=pltpu.CompilerParams(
            dimension_semantics=("arbitrary",)),
    )(hb, agg, Wn1, bn1, Wn2, bn2, Wout, bout, Wc)


def kernel(node_h, coords, edge_index, edge_feat,
           W_in, b_in,
           W_e1_0, b_e1_0, W_e2_0, b_e2_0, W_n1_0, b_n1_0, W_n2_0, b_n2_0,
           W_e1_1, b_e1_1, W_e2_1, b_e2_1, W_n1_1, b_n1_1, W_n2_1, b_n2_1,
           W_out, b_out, Wc):
    src2 = edge_index[0].reshape(NG, G)
    dst2 = edge_index[1].reshape(NG, G)

    r1 = lambda b: b.reshape(1, C)
    We1hh = (W_e1_0[:2 * C], W_e1_1[:2 * C])
    wc = (W_e1_0[2 * C:2 * C + 1], W_e1_1[2 * C:2 * C + 1])
    Wf = (W_e1_0[2 * C + 1:], W_e1_1[2 * C + 1:])
    b1 = (r1(b_e1_0), r1(b_e1_1))
    We2 = (W_e2_0, W_e2_1)
    b2 = (r1(b_e2_0), r1(b_e2_1))
    Wn1 = (W_n1_0, W_n1_1)
    bn1 = (r1(b_n1_0), r1(b_n1_1))
    Wn2 = (W_n2_0, W_n2_1)
    bn2 = (r1(b_n2_0), r1(b_n2_1))

    hb, cp = _tc_pro(node_h, coords, W_in, r1(b_in))
    sqd = _sc_sqd(cp, src2, dst2)
    sq2 = sqd.reshape(E, 1)

    hd, hs = _sc_gather(hb, src2, dst2)
    m = _tc_mid(hd, hs, sq2, edge_feat, We1hh[0], wc[0], Wf[0], b1[0],
                We2[0], b2[0])
    agg = _sc_scatter(m, dst2)
    hb = _tc_node(hb, agg, Wn1[0], bn1[0], Wn2[0], bn2[0])

    hd, hs = _sc_gather(hb, src2, dst2)
    m = _tc_mid(hd, hs, sq2, edge_feat, We1hh[1], wc[1], Wf[1], b1[1],
                We2[1], b2[1])
    agg = _sc_scatter(m, dst2)
    hs0, cs = _tc_fin(hb, agg, Wn1[1], bn1[1], Wn2[1], bn2[1],
                      W_out, r1(b_out), Wc)
    return hs0, cs


# speedup vs baseline: 3.6493x; 1.0230x over previous
import functools

import jax
import jax.numpy as jnp
from jax import lax
from jax.experimental import pallas as pl
from jax.experimental.pallas import tpu as pltpu
from jax.experimental.pallas import tpu_sc as plsc

N = 100000
E = 1600000
C = 32
D_IN = 102
G = 128
NG = E // G
NC, NS = 2, 16
NW = NC * NS
GP_W = NG // NW
GP_W_REM = NG - GP_W * NW
GP_S = NG // NS
GP_S_REM = NG - GP_S * NS
RPS = N // NS
ZR = 125
KB = 8
NBW = GP_W // KB
NBS = GP_S // KB

_f32 = jnp.float32
_bf16 = jnp.bfloat16

_SC_PARAMS = pltpu.CompilerParams(
    needs_layout_passes=False, use_tc_tiling_on_sc=False)


@functools.cache
def _mesh():
    return plsc.VectorSubcoreMesh(
        core_axis_name="c", subcore_axis_name="s",
        num_cores=NC, num_subcores=NS)


def _silu(x):
    return x * jax.nn.sigmoid(x)


def _bdot(a, b):
    return jnp.dot(a.astype(_bf16), b.astype(_bf16),
                   preferred_element_type=_f32)


def _sqd_body(cp_hbm, src2_hbm, dst2_hbm, sqd_hbm,
              idxd_v, idxs_v, rd_v, rs_v, out_v, semd, sems):
    wid = lax.axis_index("s") * NC + lax.axis_index("c")
    base = wid * GP_W + jnp.minimum(wid, GP_W_REM)
    ng = GP_W + jnp.where(wid < GP_W_REM, 1, 0)
    lanes = lax.iota(jnp.int32, 16)

    def _compute(b):
        for t in range(8):
            rows = b * G + t * 16 + lanes
            acc = jnp.zeros((16,), _f32)
            for l in range(3):
                col = jnp.full((16,), l, jnp.int32)
                vd = plsc.load_gather(rd_v, [rows, col])
                vs = plsc.load_gather(rs_v, [rows, col])
                d = vd - vs
                acc = acc + d * d
            out_v[pl.ds(b * G + t * 16, 16)] = acc

    @pl.loop(0, NBW)
    def _(i):
        gb = base + i * KB
        pltpu.sync_copy(dst2_hbm.at[pl.ds(gb, KB), :], idxd_v)
        pltpu.sync_copy(src2_hbm.at[pl.ds(gb, KB), :], idxs_v)
        ds_ = []
        for b in range(KB):
            ds_.append(pltpu.async_copy(
                cp_hbm.at[idxd_v.at[b]],
                rd_v.at[pl.ds(b * G, G), :], semd))
            ds_.append(pltpu.async_copy(
                cp_hbm.at[idxs_v.at[b]],
                rs_v.at[pl.ds(b * G, G), :], sems))
        for d in ds_:
            d.wait()
        for b in range(KB):
            _compute(b)
        pltpu.sync_copy(out_v, sqd_hbm.at[pl.ds(gb * G, KB * G)])

    @pl.loop(NBW * KB, ng)
    def _(j):
        g = base + j
        pltpu.sync_copy(dst2_hbm.at[g], idxd_v.at[0])
        pltpu.sync_copy(src2_hbm.at[g], idxs_v.at[0])
        cd = pltpu.async_copy(cp_hbm.at[idxd_v.at[0]],
                              rd_v.at[pl.ds(0, G), :], semd)
        cs = pltpu.async_copy(cp_hbm.at[idxs_v.at[0]],
                              rs_v.at[pl.ds(0, G), :], sems)
        cd.wait()
        cs.wait()
        _compute(0)
        pltpu.sync_copy(out_v.at[pl.ds(0, G)], sqd_hbm.at[pl.ds(g * G, G)])


@jax.jit
def _sc_sqd(cp, src2, dst2):
    return pl.kernel(
        _sqd_body,
        out_type=jax.ShapeDtypeStruct((E,), _f32),
        mesh=_mesh(),
        scratch_types=[
            pltpu.VMEM((KB, G), jnp.int32),
            pltpu.VMEM((KB, G), jnp.int32),
            pltpu.VMEM((KB * G, 16), _f32),
            pltpu.VMEM((KB * G, 16), _f32),
            pltpu.VMEM((KB * G,), _f32),
            pltpu.SemaphoreType.DMA,
            pltpu.SemaphoreType.DMA,
        ],
        compiler_params=_SC_PARAMS,
    )(cp, src2, dst2)


def _gath_body(hb_hbm, src2_hbm, dst2_hbm, hd_hbm, hs_hbm,
               idxd_v, idxs_v, bufd_v, bufs_v, semd, sems):
    wid = lax.axis_index("s") * NC + lax.axis_index("c")
    base = wid * GP_W + jnp.minimum(wid, GP_W_REM)
    ng = GP_W + jnp.where(wid < GP_W_REM, 1, 0)

    @pl.loop(0, NBW)
    def _(i):
        gb = base + i * KB
        pltpu.sync_copy(dst2_hbm.at[pl.ds(gb, KB), :], idxd_v)
        pltpu.sync_copy(src2_hbm.at[pl.ds(gb, KB), :], idxs_v)
        ds_ = []
        for b in range(KB):
            ds_.append(pltpu.async_copy(
                hb_hbm.at[idxd_v.at[b]],
                bufd_v.at[pl.ds(b * G, G), :], semd))
            ds_.append(pltpu.async_copy(
                hb_hbm.at[idxs_v.at[b]],
                bufs_v.at[pl.ds(b * G, G), :], sems))
        for d in ds_:
            d.wait()
        s1 = pltpu.async_copy(bufd_v, hd_hbm.at[pl.ds(gb * G, KB * G), :],
                              semd)
        s2 = pltpu.async_copy(bufs_v, hs_hbm.at[pl.ds(gb * G, KB * G), :],
                              sems)
        s1.wait()
        s2.wait()

    @pl.loop(NBW * KB, ng)
    def _(j):
        g = base + j
        pltpu.sync_copy(dst2_hbm.at[g], idxd_v.at[0])
        pltpu.sync_copy(src2_hbm.at[g], idxs_v.at[0])
        cd = pltpu.async_copy(hb_hbm.at[idxd_v.at[0]],
                              bufd_v.at[pl.ds(0, G), :], semd)
        cs = pltpu.async_copy(hb_hbm.at[idxs_v.at[0]],
                              bufs_v.at[pl.ds(0, G), :], sems)
        cd.wait()
        cs.wait()
        pltpu.sync_copy(bufd_v.at[pl.ds(0, G), :],
                        hd_hbm.at[pl.ds(g * G, G), :])
        pltpu.sync_copy(bufs_v.at[pl.ds(0, G), :],
                        hs_hbm.at[pl.ds(g * G, G), :])


@jax.jit
def _sc_gather(hb, src2, dst2):
    return pl.kernel(
        _gath_body,
        out_type=(jax.ShapeDtypeStruct((E, C), _bf16),
                  jax.ShapeDtypeStruct((E, C), _bf16)),
        mesh=_mesh(),
        scratch_types=[
            pltpu.VMEM((KB, G), jnp.int32),
            pltpu.VMEM((KB, G), jnp.int32),
            pltpu.VMEM((KB * G, C), _bf16),
            pltpu.VMEM((KB * G, C), _bf16),
            pltpu.SemaphoreType.DMA,
            pltpu.SemaphoreType.DMA,
        ],
        compiler_params=_SC_PARAMS,
    )(hb, src2, dst2)


def _scat_body(m_hbm, dst2_hbm, agg_hbm, idx_v, mb_v, zb_v, acc_sh, sem):
    c = lax.axis_index("c")
    s = lax.axis_index("s")
    cc = c * 16

    @pl.loop(0, ZR)
    def _(r):
        zb_v[r, :] = jnp.zeros((16,), _f32)

    @pl.loop(0, RPS // ZR)
    def _(j):
        pltpu.sync_copy(zb_v, acc_sh.at[pl.ds(s * RPS + j * ZR, ZR), :])

    plsc.subcore_barrier()

    base = s * GP_S + jnp.minimum(s, GP_S_REM)
    ng = GP_S + jnp.where(s < GP_S_REM, 1, 0)

    @pl.loop(0, NBS)
    def _(i):
        gb = base + i * KB
        pltpu.sync_copy(dst2_hbm.at[pl.ds(gb, KB), :], idx_v)
        pltpu.sync_copy(m_hbm.at[pl.ds(gb * G, KB * G), pl.ds(cc, 16)],
                        mb_v)
        ds_ = []
        for b in range(KB):
            ds_.append(pltpu.async_copy(
                mb_v.at[pl.ds(b * G, G), :],
                acc_sh.at[idx_v.at[b]], sem, add=True))
        for d in ds_:
            d.wait()

    @pl.loop(NBS * KB, ng)
    def _(j):
        g = base + j
        pltpu.sync_copy(dst2_hbm.at[g], idx_v.at[0])
        pltpu.sync_copy(m_hbm.at[pl.ds(g * G, G), pl.ds(cc, 16)],
                        mb_v.at[pl.ds(0, G), :])
        pltpu.sync_copy(mb_v.at[pl.ds(0, G), :], acc_sh.at[idx_v.at[0]],
                        add=True)

    plsc.subcore_barrier()

    @pl.loop(0, RPS // ZR)
    def _(j):
        r0 = s * RPS + j * ZR
        pltpu.sync_copy(acc_sh.at[pl.ds(r0, ZR), :],
                        agg_hbm.at[pl.ds(r0, ZR), pl.ds(cc, 16)])


@jax.jit
def _sc_scatter(m, dst2):
    return pl.kernel(
        _scat_body,
        out_type=jax.ShapeDtypeStruct((N, C), _f32),
        mesh=_mesh(),
        scratch_types=[
            pltpu.VMEM((KB, G), jnp.int32),
            pltpu.VMEM((KB * G, 16), _f32),
            pltpu.VMEM((ZR, 16), _f32),
            pltpu.VMEM_SHARED((N, 16), _f32),
            pltpu.SemaphoreType.DMA,
        ],
        compiler_params=_SC_PARAMS,
    )(m, dst2)


BN = 1000
BE = 4000


def _full_spec(shape):
    return pl.BlockSpec(shape, lambda i: tuple(0 for _ in shape))


def _row_spec(rows, cols):
    return pl.BlockSpec((rows, cols), lambda i: (i, 0))


def _pro_body(nh, co, Win, bin_, hb_o, cp_o):
    h = _silu(_bdot(nh[...], Win[...]) + bin_[...])
    hb_o[...] = h.astype(_bf16)
    cp_o[...] = jnp.concatenate(
        [co[...], jnp.zeros((BN, 13), _f32)], axis=1)


@jax.jit
def _tc_pro(node_h, coords, W_in, b_in):
    return pl.pallas_call(
        _pro_body,
        grid=(N // BN,),
        in_specs=[
            _row_spec(BN, D_IN), _row_spec(BN, 3),
            _full_spec((D_IN, C)), _full_spec((1, C)),
        ],
        out_specs=[_row_spec(BN, C), _row_spec(BN, 16)],
        out_shape=[jax.ShapeDtypeStruct((N, C), _bf16),
                   jax.ShapeDtypeStruct((N, 16), _f32)],
        compiler_params=pltpu.CompilerParams(
            dimension_semantics=("arbitrary",)),
    )(node_h, coords, W_in, b_in)


def _mid_body(hd, hs, sq, ef, We1hh, wc, Wf, b1, We2, b2, m_o):
    x = jnp.concatenate([hd[...], hs[...]], axis=1)
    sqb = sq[...].astype(_f32)
    wcb = wc[...].astype(_bf16).astype(_f32)
    a = (jnp.dot(x, We1hh[...].astype(_bf16), preferred_element_type=_f32)
         + sqb * wcb
         + _bdot(ef[...], Wf[...])
         + b1[...])
    m1 = _silu(a)
    z = _bdot(m1, We2[...]) + b2[...]
    m_o[...] = _silu(z)


@jax.jit
def _tc_mid(hd, hs, sq2, ef, We1hh, wc, Wf, b1, We2, b2):
    return pl.pallas_call(
        _mid_body,
        grid=(E // BE,),
        in_specs=[
            _row_spec(BE, C), _row_spec(BE, C), _row_spec(BE, 1),
            _row_spec(BE, 3),
            _full_spec((2 * C, C)), _full_spec((1, C)), _full_spec((3, C)),
            _full_spec((1, C)),
            _full_spec((C, C)), _full_spec((1, C)),
        ],
        out_specs=[_row_spec(BE, C)],
        out_shape=[jax.ShapeDtypeStruct((E, C), _f32)],
        compiler_params=pltpu.CompilerParams(
            dimension_semantics=("arbitrary",)),
    )(hd, hs, sq2, ef, We1hh, wc, Wf, b1, We2, b2)[0]


def _node_body(hb, agg, Wn1, bn1, Wn2, bn2, hb_o):
    hn = jnp.concatenate([hb[...], agg[...].astype(_bf16)], axis=1)
    t = jnp.dot(hn, Wn1[...].astype(_bf16),
                preferred_element_type=_f32) + bn1[...]
    h2 = _bdot(_silu(t), Wn2[...]) + bn2[...]
    hb_o[...] = h2.astype(_bf16)


@jax.jit
def _tc_node(hb, agg, Wn1, bn1, Wn2, bn2):
    return pl.pallas_call(
        _node_body,
        grid=(N // BN,),
        in_specs=[
            _row_spec(BN, C), _row_spec(BN, C),
            _full_spec((2 * C, C)), _full_spec((1, C)),
            _full_spec((C, C)), _full_spec((1, C)),
        ],
        out_specs=[_row_spec(BN, C)],
        out_shape=[jax.ShapeDtypeStruct((N, C), _bf16)],
        compiler_params=pltpu.CompilerParams(
            dimension_semantics=("arbitrary",)),
    )(hb, agg, Wn1, bn1, Wn2, bn2)[0]


def _fin_body(hb, agg, Wn1, bn1, Wn2, bn2, Wout, bout, Wc, hs_o, cs_o):
    hn = jnp.concatenate([hb[...], agg[...].astype(_bf16)], axis=1)
    t = jnp.dot(hn, Wn1[...].astype(_bf16),
                preferred_element_type=_f32) + bn1[...]
    h2 = _bdot(_silu(t), Wn2[...]) + bn2[...]
    hs = _silu(_bdot(h2, Wout[...]) + bout[...])
    hs_o[...] = hs
    cs_o[...] = _bdot(hs, Wc[...])


@jax.jit
def _tc_fin(hb, agg, Wn1, bn1, Wn2, bn2, Wout, bout, Wc):
    return pl.pallas_call(
        _fin_body,
        grid=(N // BN,),
        in_specs=[
            _row_spec(BN, C), _row_spec(BN, C),
            _full_spec((2 * C, C)), _full_spec((1, C)),
            _full_spec((C, C)), _full_spec((1, C)),
            _full_spec((C, C)), _full_spec((1, C)),
            _full_spec((C, 6)),
        ],
        out_specs=[_row_spec(BN, C), _row_spec(BN, 6)],
        out_shape=[jax.ShapeDtypeStruct((N, C), _f32),
                   jax.ShapeDtypeStruct((N, 6), _f32)],
        compiler_params=pltpu.CompilerParams(
            dimension_semantics=("arbitrary",)),
    )(hb, agg, Wn1, bn1, Wn2, bn2, Wout, bout, Wc)


def kernel(node_h, coords, edge_index, edge_feat,
           W_in, b_in,
           W_e1_0, b_e1_0, W_e2_0, b_e2_0, W_n1_0, b_n1_0, W_n2_0, b_n2_0,
           W_e1_1, b_e1_1, W_e2_1, b_e2_1, W_n1_1, b_n1_1, W_n2_1, b_n2_1,
           W_out, b_out, Wc):
    src2 = edge_index[0].reshape(NG, G)
    dst2 = edge_index[1].reshape(NG, G)

    r1 = lambda b: b.reshape(1, C)
    We1hh = (W_e1_0[:2 * C], W_e1_1[:2 * C])
    wc = (W_e1_0[2 * C:2 * C + 1], W_e1_1[2 * C:2 * C + 1])
    Wf = (W_e1_0[2 * C + 1:], W_e1_1[2 * C + 1:])
    b1 = (r1(b_e1_0), r1(b_e1_1))
    We2 = (W_e2_0, W_e2_1)
    b2 = (r1(b_e2_0), r1(b_e2_1))
    Wn1 = (W_n1_0, W_n1_1)
    bn1 = (r1(b_n1_0), r1(b_n1_1))
    Wn2 = (W_n2_0, W_n2_1)
    bn2 = (r1(b_n2_0), r1(b_n2_1))

    hb, cp = _tc_pro(node_h, coords, W_in, r1(b_in))
    sqd = _sc_sqd(cp, src2, dst2)
    sq2 = sqd.astype(_bf16).reshape(E, 1)
    edge_feat = edge_feat.astype(_bf16)

    hd, hs = _sc_gather(hb, src2, dst2)
    m = _tc_mid(hd, hs, sq2, edge_feat, We1hh[0], wc[0], Wf[0], b1[0],
                We2[0], b2[0])
    agg = _sc_scatter(m, dst2)
    hb = _tc_node(hb, agg, Wn1[0], bn1[0], Wn2[0], bn2[0])

    hd, hs = _sc_gather(hb, src2, dst2)
    m = _tc_mid(hd, hs, sq2, edge_feat, We1hh[1], wc[1], Wf[1], b1[1],
                We2[1], b2[1])
    agg = _sc_scatter(m, dst2)
    hs0, cs = _tc_fin(hb, agg, Wn1[1], bn1[1], Wn2[1], bn2[1],
                      W_out, r1(b_out), Wc)
    return hs0, cs
